# baseline (speedup 1.0000x reference)
import functools

import jax
import jax.numpy as jnp
from jax import lax
from jax.experimental import pallas as pl
from jax.experimental.pallas import tpu as pltpu
from jax.experimental.pallas import tpu_sc as plsc

BATCH = 16384
EMB = 64
FEAT = 128
NC, NS = 2, 16
NW = NC * NS
B_PER_W = BATCH // NW
B_CHUNK = 256

_MESH = plsc.VectorSubcoreMesh(core_axis_name="c", subcore_axis_name="s")


@jax.jit
def _sc_gather_both(user_pairs, item_pairs, user_ids, item_ids):
    @functools.partial(
        pl.kernel,
        mesh=_MESH,
        out_type=[
            jax.ShapeDtypeStruct((BATCH, 2 * EMB), jnp.float32),
            jax.ShapeDtypeStruct((BATCH, 2 * EMB), jnp.float32),
        ],
        scratch_types=[
            pltpu.VMEM((B_CHUNK,), jnp.int32),
            pltpu.VMEM((B_CHUNK, 2 * EMB), jnp.float32),
            pltpu.VMEM((B_CHUNK,), jnp.int32),
            pltpu.VMEM((B_CHUNK, 2 * EMB), jnp.float32),
            pltpu.SemaphoreType.DMA,
            pltpu.SemaphoreType.DMA,
        ],
    )
    def k(ut_hbm, it_hbm, uid_hbm, iid_hbm, uout_hbm, iout_hbm,
          uidx_v, urows_v, iidx_v, irows_v, usem, isem):
        wid = lax.axis_index("s") * NC + lax.axis_index("c")
        for chunk in range(B_PER_W // B_CHUNK):
            base = wid * B_PER_W + chunk * B_CHUNK
            pltpu.sync_copy(uid_hbm.at[pl.ds(base, B_CHUNK)], uidx_v)
            pltpu.sync_copy(iid_hbm.at[pl.ds(base, B_CHUNK)], iidx_v)
            uc = pltpu.async_copy(ut_hbm.at[uidx_v], urows_v, usem)
            ic = pltpu.async_copy(it_hbm.at[iidx_v], irows_v, isem)
            uc.wait()
            ic.wait()
            pltpu.sync_copy(urows_v, uout_hbm.at[pl.ds(base, B_CHUNK)])
            pltpu.sync_copy(irows_v, iout_hbm.at[pl.ds(base, B_CHUNK)])

    return k(user_pairs, item_pairs, user_ids, item_ids)


def _mlp_body(uf_ref, wm_ref, wfh_ref, bm_ref, bf_ref, out_ref):
    h = jnp.dot(uf_ref[...], wm_ref[...], preferred_element_type=jnp.float32)
    h = jnp.maximum(h + bm_ref[...], 0.0)
    out_ref[...] = (
        jnp.dot(h, wfh_ref[...], preferred_element_type=jnp.float32) + bf_ref[...]
    )


def _final_body(gu_ref, gi_ref, uid_ref, iid_ref, t_ref, wfu_ref, out_ref):
    upar = (uid_ref[...] % 2 == 0).reshape(-1, 1)
    ipar = (iid_ref[...] % 2 == 0).reshape(-1, 1)
    ue = jnp.where(upar, gu_ref[:, :EMB], gu_ref[:, EMB:])
    ie = jnp.where(ipar, gi_ref[:, :EMB], gi_ref[:, EMB:])
    fused = (
        jnp.dot(ue, wfu_ref[...], preferred_element_type=jnp.float32)
        + t_ref[...]
    )
    out_ref[...] = jnp.sum(fused * ie, axis=1)


def kernel(user_ids, item_ids, user_features, user_table, item_table,
           W_mlp, b_mlp, W_fuse, b_fuse):
    user_ids = user_ids.astype(jnp.int32)
    item_ids = item_ids.astype(jnp.int32)

    wm_t = W_mlp.T
    wfu = W_fuse[:, :EMB].T
    wfh = W_fuse[:, EMB:].T
    bm = b_mlp.reshape(1, EMB)
    bf = b_fuse.reshape(1, EMB)

    BM = 2048
    grid = (BATCH // BM,)

    t = pl.pallas_call(
        _mlp_body,
        grid=grid,
        in_specs=[
            pl.BlockSpec((BM, FEAT), lambda i: (i, 0)),
            pl.BlockSpec((FEAT, EMB), lambda i: (0, 0)),
            pl.BlockSpec((EMB, EMB), lambda i: (0, 0)),
            pl.BlockSpec((1, EMB), lambda i: (0, 0)),
            pl.BlockSpec((1, EMB), lambda i: (0, 0)),
        ],
        out_specs=pl.BlockSpec((BM, EMB), lambda i: (i, 0)),
        out_shape=jax.ShapeDtypeStruct((BATCH, EMB), jnp.float32),
    )(user_features, wm_t, wfh, bm, bf)

    user_pairs = user_table.reshape(-1, 2 * EMB)
    item_pairs = item_table.reshape(-1, 2 * EMB)
    gu, gi = _sc_gather_both(user_pairs, item_pairs,
                             user_ids // 2, item_ids // 2)

    uid2 = user_ids.reshape(BATCH, 1)
    iid2 = item_ids.reshape(BATCH, 1)
    out = pl.pallas_call(
        _final_body,
        grid=grid,
        in_specs=[
            pl.BlockSpec((BM, 2 * EMB), lambda i: (i, 0)),
            pl.BlockSpec((BM, 2 * EMB), lambda i: (i, 0)),
            pl.BlockSpec((BM, 1), lambda i: (i, 0)),
            pl.BlockSpec((BM, 1), lambda i: (i, 0)),
            pl.BlockSpec((BM, EMB), lambda i: (i, 0)),
            pl.BlockSpec((EMB, EMB), lambda i: (0, 0)),
        ],
        out_specs=pl.BlockSpec((BM,), lambda i: (i,)),
        out_shape=jax.ShapeDtypeStruct((BATCH,), jnp.float32),
    )(gu, gi, uid2, iid2, t, wfu)

    return out


# speedup vs baseline: 1.0048x; 1.0048x over previous
import functools

import jax
import jax.numpy as jnp
from jax import lax
from jax.experimental import pallas as pl
from jax.experimental.pallas import tpu as pltpu
from jax.experimental.pallas import tpu_sc as plsc

BATCH = 16384
EMB = 64
FEAT = 128
NC, NS = 2, 16
NW = NC * NS
B_PER_W = BATCH // NW

_MESH = plsc.VectorSubcoreMesh(core_axis_name="c", subcore_axis_name="s")


@jax.jit
def _sc_gather_both(user_table, item_table, user_ids, item_ids):
    @functools.partial(
        pl.kernel,
        mesh=_MESH,
        out_type=[
            jax.ShapeDtypeStruct((BATCH, EMB), jnp.float32),
            jax.ShapeDtypeStruct((BATCH, EMB), jnp.float32),
        ],
        scratch_types=[
            pltpu.VMEM((B_PER_W,), jnp.int32),
            pltpu.VMEM((B_PER_W, EMB), jnp.float32),
            pltpu.VMEM((B_PER_W,), jnp.int32),
            pltpu.VMEM((B_PER_W, EMB), jnp.float32),
            pltpu.SemaphoreType.DMA,
            pltpu.SemaphoreType.DMA,
        ],
        compiler_params=pltpu.CompilerParams(use_tc_tiling_on_sc=False),
    )
    def k(ut_hbm, it_hbm, uid_hbm, iid_hbm, uout_hbm, iout_hbm,
          uidx_v, urows_v, iidx_v, irows_v, usem, isem):
        wid = lax.axis_index("s") * NC + lax.axis_index("c")
        base = wid * B_PER_W
        pltpu.sync_copy(uid_hbm.at[pl.ds(base, B_PER_W)], uidx_v)
        pltpu.sync_copy(iid_hbm.at[pl.ds(base, B_PER_W)], iidx_v)
        uc = pltpu.async_copy(ut_hbm.at[uidx_v], urows_v, usem)
        ic = pltpu.async_copy(it_hbm.at[iidx_v], irows_v, isem)
        uc.wait()
        ic.wait()
        pltpu.sync_copy(urows_v, uout_hbm.at[pl.ds(base, B_PER_W)])
        pltpu.sync_copy(irows_v, iout_hbm.at[pl.ds(base, B_PER_W)])

    return k(user_table, item_table, user_ids, item_ids)


def _mlp_body(uf_ref, wm_ref, wfh_ref, bm_ref, bf_ref, out_ref):
    h = jnp.dot(uf_ref[...], wm_ref[...], preferred_element_type=jnp.float32)
    h = jnp.maximum(h + bm_ref[...], 0.0)
    out_ref[...] = (
        jnp.dot(h, wfh_ref[...], preferred_element_type=jnp.float32) + bf_ref[...]
    )


def _final_body(ue_ref, ie_ref, t_ref, wfu_ref, out_ref):
    fused = (
        jnp.dot(ue_ref[...], wfu_ref[...], preferred_element_type=jnp.float32)
        + t_ref[...]
    )
    out_ref[...] = jnp.sum(fused * ie_ref[...], axis=1)


def kernel(user_ids, item_ids, user_features, user_table, item_table,
           W_mlp, b_mlp, W_fuse, b_fuse):
    user_ids = user_ids.astype(jnp.int32)
    item_ids = item_ids.astype(jnp.int32)

    wm_t = W_mlp.T
    wfu = W_fuse[:, :EMB].T
    wfh = W_fuse[:, EMB:].T
    bm = b_mlp.reshape(1, EMB)
    bf = b_fuse.reshape(1, EMB)

    BM = 2048
    grid = (BATCH // BM,)

    t = pl.pallas_call(
        _mlp_body,
        grid=grid,
        in_specs=[
            pl.BlockSpec((BM, FEAT), lambda i: (i, 0)),
            pl.BlockSpec((FEAT, EMB), lambda i: (0, 0)),
            pl.BlockSpec((EMB, EMB), lambda i: (0, 0)),
            pl.BlockSpec((1, EMB), lambda i: (0, 0)),
            pl.BlockSpec((1, EMB), lambda i: (0, 0)),
        ],
        out_specs=pl.BlockSpec((BM, EMB), lambda i: (i, 0)),
        out_shape=jax.ShapeDtypeStruct((BATCH, EMB), jnp.float32),
    )(user_features, wm_t, wfh, bm, bf)

    user_emb, item_emb = _sc_gather_both(user_table, item_table,
                                         user_ids, item_ids)

    out = pl.pallas_call(
        _final_body,
        grid=grid,
        in_specs=[
            pl.BlockSpec((BM, EMB), lambda i: (i, 0)),
            pl.BlockSpec((BM, EMB), lambda i: (i, 0)),
            pl.BlockSpec((BM, EMB), lambda i: (i, 0)),
            pl.BlockSpec((EMB, EMB), lambda i: (0, 0)),
        ],
        out_specs=pl.BlockSpec((BM,), lambda i: (i,)),
        out_shape=jax.ShapeDtypeStruct((BATCH,), jnp.float32),
    )(user_emb, item_emb, t, wfu)

    return out


# speedup vs baseline: 1.0057x; 1.0008x over previous
import functools

import jax
import jax.numpy as jnp
from jax import lax
from jax.experimental import pallas as pl
from jax.experimental.pallas import tpu as pltpu
from jax.experimental.pallas import tpu_sc as plsc

BATCH = 16384
EMB = 64
FEAT = 128
NC, NS = 2, 16
NW = NC * NS
B_PER_W = BATCH // NW

_MESH = plsc.VectorSubcoreMesh(core_axis_name="c", subcore_axis_name="s")


def _sc_gather_one(table, ids):
    @functools.partial(
        pl.kernel,
        mesh=_MESH,
        out_type=jax.ShapeDtypeStruct((BATCH, EMB), jnp.float32),
        scratch_types=[
            pltpu.VMEM((B_PER_W,), jnp.int32),
            pltpu.VMEM((B_PER_W, EMB), jnp.float32),
            pltpu.SemaphoreType.DMA,
        ],
        compiler_params=pltpu.CompilerParams(use_tc_tiling_on_sc=False),
    )
    def k(t_hbm, id_hbm, out_hbm, idx_v, rows_v, sem):
        wid = lax.axis_index("s") * NC + lax.axis_index("c")
        base = wid * B_PER_W
        pltpu.sync_copy(id_hbm.at[pl.ds(base, B_PER_W)], idx_v)
        pltpu.async_copy(t_hbm.at[idx_v], rows_v, sem).wait()
        pltpu.sync_copy(rows_v, out_hbm.at[pl.ds(base, B_PER_W)])

    return k(table, ids)


def _mlp_body(uf_ref, wm_ref, wfh_ref, bm_ref, bf_ref, out_ref):
    h = jnp.dot(uf_ref[...], wm_ref[...], preferred_element_type=jnp.float32)
    h = jnp.maximum(h + bm_ref[...], 0.0)
    out_ref[...] = (
        jnp.dot(h, wfh_ref[...], preferred_element_type=jnp.float32) + bf_ref[...]
    )


def _final_body(ue_ref, ie_ref, t_ref, wfu_ref, out_ref):
    fused = (
        jnp.dot(ue_ref[...], wfu_ref[...], preferred_element_type=jnp.float32)
        + t_ref[...]
    )
    out_ref[...] = jnp.sum(fused * ie_ref[...], axis=1)


def kernel(user_ids, item_ids, user_features, user_table, item_table,
           W_mlp, b_mlp, W_fuse, b_fuse):
    user_ids = user_ids.astype(jnp.int32)
    item_ids = item_ids.astype(jnp.int32)

    wm_t = W_mlp.T
    wfu = W_fuse[:, :EMB].T
    wfh = W_fuse[:, EMB:].T
    bm = b_mlp.reshape(1, EMB)
    bf = b_fuse.reshape(1, EMB)

    BM = 2048
    grid = (BATCH // BM,)

    t = pl.pallas_call(
        _mlp_body,
        grid=grid,
        in_specs=[
            pl.BlockSpec((BM, FEAT), lambda i: (i, 0)),
            pl.BlockSpec((FEAT, EMB), lambda i: (0, 0)),
            pl.BlockSpec((EMB, EMB), lambda i: (0, 0)),
            pl.BlockSpec((1, EMB), lambda i: (0, 0)),
            pl.BlockSpec((1, EMB), lambda i: (0, 0)),
        ],
        out_specs=pl.BlockSpec((BM, EMB), lambda i: (i, 0)),
        out_shape=jax.ShapeDtypeStruct((BATCH, EMB), jnp.float32),
    )(user_features, wm_t, wfh, bm, bf)

    user_emb = _sc_gather_one(user_table, user_ids)
    item_emb = _sc_gather_one(item_table, item_ids)

    out = pl.pallas_call(
        _final_body,
        grid=grid,
        in_specs=[
            pl.BlockSpec((BM, EMB), lambda i: (i, 0)),
            pl.BlockSpec((BM, EMB), lambda i: (i, 0)),
            pl.BlockSpec((BM, EMB), lambda i: (i, 0)),
            pl.BlockSpec((EMB, EMB), lambda i: (0, 0)),
        ],
        out_specs=pl.BlockSpec((BM,), lambda i: (i,)),
        out_shape=jax.ShapeDtypeStruct((BATCH,), jnp.float32),
    )(user_emb, item_emb, t, wfu)

    return out


# speedup vs baseline: 1.0842x; 1.0781x over previous
import functools

import jax
import jax.numpy as jnp
from jax import lax
from jax.experimental import pallas as pl
from jax.experimental.pallas import tpu as pltpu
from jax.experimental.pallas import tpu_sc as plsc

BATCH = 16384
EMB = 64
FEAT = 128
NC, NS = 2, 16
NW = NC * NS
B_PER_W = BATCH // NW
B_CHUNK = 512

_MESH = plsc.VectorSubcoreMesh(core_axis_name="c", subcore_axis_name="s")


def _sc_gather_wide(table128, ids):
    @functools.partial(
        pl.kernel,
        mesh=_MESH,
        out_type=jax.ShapeDtypeStruct((BATCH, 2 * EMB), jnp.float32),
        scratch_types=[
            pltpu.VMEM((B_CHUNK,), jnp.int32),
            pltpu.VMEM((B_CHUNK, 2 * EMB), jnp.float32),
            pltpu.SemaphoreType.DMA,
        ],
    )
    def k(t_hbm, id_hbm, out_hbm, idx_v, rows_v, sem):
        wid = lax.axis_index("s") * NC + lax.axis_index("c")
        for chunk in range(B_PER_W // B_CHUNK):
            base = wid * B_PER_W + chunk * B_CHUNK
            pltpu.sync_copy(id_hbm.at[pl.ds(base, B_CHUNK)], idx_v)
            pltpu.async_copy(t_hbm.at[idx_v], rows_v, sem).wait()
            pltpu.sync_copy(rows_v, out_hbm.at[pl.ds(base, B_CHUNK)])

    return k(table128, ids)


def _mlp_body(uf_ref, wm_ref, wfh_ref, bm_ref, bf_ref, out_ref):
    h = jnp.dot(uf_ref[...], wm_ref[...], preferred_element_type=jnp.float32)
    h = jnp.maximum(h + bm_ref[...], 0.0)
    out_ref[...] = (
        jnp.dot(h, wfh_ref[...], preferred_element_type=jnp.float32) + bf_ref[...]
    )


def _final_body(ue_ref, ie_ref, t_ref, wfu_ref, out_ref):
    ue = ue_ref[:, :EMB].astype(jnp.float32)
    ie = ie_ref[:, :EMB].astype(jnp.float32)
    fused = (
        jnp.dot(ue, wfu_ref[...], preferred_element_type=jnp.float32)
        + t_ref[...]
    )
    out_ref[...] = jnp.sum(fused * ie, axis=1)


def kernel(user_ids, item_ids, user_features, user_table, item_table,
           W_mlp, b_mlp, W_fuse, b_fuse):
    user_ids = user_ids.astype(jnp.int32)
    item_ids = item_ids.astype(jnp.int32)

    wm_t = W_mlp.T
    wfu = W_fuse[:, :EMB].T
    wfh = W_fuse[:, EMB:].T
    bm = b_mlp.reshape(1, EMB)
    bf = b_fuse.reshape(1, EMB)

    BM = 2048
    grid = (BATCH // BM,)

    t = pl.pallas_call(
        _mlp_body,
        grid=grid,
        in_specs=[
            pl.BlockSpec((BM, FEAT), lambda i: (i, 0)),
            pl.BlockSpec((FEAT, EMB), lambda i: (0, 0)),
            pl.BlockSpec((EMB, EMB), lambda i: (0, 0)),
            pl.BlockSpec((1, EMB), lambda i: (0, 0)),
            pl.BlockSpec((1, EMB), lambda i: (0, 0)),
        ],
        out_specs=pl.BlockSpec((BM, EMB), lambda i: (i, 0)),
        out_shape=jax.ShapeDtypeStruct((BATCH, EMB), jnp.float32),
    )(user_features, wm_t, wfh, bm, bf)

    user_emb = _sc_gather_wide(jnp.pad(user_table, ((0, 0), (0, EMB))),
                               user_ids)
    item_emb = _sc_gather_wide(jnp.pad(item_table, ((0, 0), (0, EMB))),
                               item_ids)

    out = pl.pallas_call(
        _final_body,
        grid=grid,
        in_specs=[
            pl.BlockSpec((BM, 2 * EMB), lambda i: (i, 0)),
            pl.BlockSpec((BM, 2 * EMB), lambda i: (i, 0)),
            pl.BlockSpec((BM, EMB), lambda i: (i, 0)),
            pl.BlockSpec((EMB, EMB), lambda i: (0, 0)),
        ],
        out_specs=pl.BlockSpec((BM,), lambda i: (i,)),
        out_shape=jax.ShapeDtypeStruct((BATCH,), jnp.float32),
    )(user_emb, item_emb, t, wfu)

    return out
